# speedup vs baseline: 6.6548x; 6.6548x over previous
import functools

import jax
import jax.numpy as jnp
from jax import lax
from jax.experimental import pallas as pl
from jax.experimental.pallas import tpu as pltpu
from jax.experimental.pallas import tpu_sc as plsc

_HIGH = lax.Precision.HIGHEST
_EPS = 1e-6

_NC = 2
_NS = 16
_NW = _NC * _NS


def _leaky(t):
    return jnp.where(t >= 0, t, 0.01 * t)


def _sigmoid(t):
    return 1.0 / (1.0 + jnp.exp(-t))


def _stats_body(G, NB, x_ref, gidr_ref, s1_ref, s2_ref, c3_ref):
    i = pl.program_id(0)
    x = x_ref[...]
    B = x.shape[0]
    gid = gidr_ref[0]
    m = (lax.broadcasted_iota(jnp.int32, (G, B), 0) == gid).astype(jnp.float32)

    @pl.when(i == 0)
    def _():
        s1_ref[...] = jnp.zeros_like(s1_ref)
        s2_ref[...] = jnp.zeros_like(s2_ref)
        c3_ref[...] = jnp.zeros_like(c3_ref)

    s1_ref[...] += jnp.dot(m, x, precision=_HIGH)
    s2_ref[...] += jnp.dot(m, x * x, precision=_HIGH)
    c3_ref[...] += jnp.dot(m, jnp.ones_like(x), precision=_HIGH)


def _stats_call(x, gid_row3, G, B, NB, D):
    return pl.pallas_call(
        functools.partial(_stats_body, G, NB),
        grid=(NB,),
        in_specs=[
            pl.BlockSpec((B, D), lambda i: (i, 0)),
            pl.BlockSpec((1, 1, B), lambda i: (i, 0, 0)),
        ],
        out_specs=[
            pl.BlockSpec((G, D), lambda i: (0, 0)),
            pl.BlockSpec((G, D), lambda i: (0, 0)),
            pl.BlockSpec((G, D), lambda i: (0, 0)),
        ],
        out_shape=[jax.ShapeDtypeStruct((G, D), jnp.float32)] * 3,
    )(x, gid_row3)


def _norm_fields(s1, s2, c3, alpha):
    cnt = jnp.maximum(c3, 1.0)
    mean = s1 / cnt
    meansq = s2 / cnt
    var = meansq - (2.0 * alpha - alpha * alpha) * mean * mean
    rstd = lax.rsqrt(var + _EPS)
    return mean, rstd


def _fuse_body(G, NB, x_ref, gidr_ref, gidc_ref, y_ref, s1_ref, s2_ref, c3_ref,
               g1_ref, b1_ref, a1_ref, ws_ref, bs_ref, wt_ref, bt_ref,
               xmid_ref, t1_ref, t2_ref):
    i = pl.program_id(0)
    x = x_ref[...]
    B, D = x.shape
    a1 = a1_ref[...]
    mean, rstd = _norm_fields(s1_ref[...], s2_ref[...], c3_ref[...], a1)
    tbl = jnp.concatenate([mean, rstd, y_ref[...]], axis=1)
    gidc = gidc_ref[0]
    mt = (gidc == lax.broadcasted_iota(jnp.int32, (B, G), 1)).astype(jnp.float32)
    nv = jnp.dot(mt, tbl, precision=_HIGH)
    mean_n = nv[:, :D]
    rstd_n = nv[:, D:2 * D]
    yn = nv[:, 2 * D:]
    x2 = g1_ref[...] * (x - a1 * mean_n) * rstd_n + b1_ref[...]
    xs = _leaky(jnp.dot(x2, ws_ref[...], precision=_HIGH) + bs_ref[...])
    xt = _leaky(jnp.dot(yn, wt_ref[...], precision=_HIGH) + bt_ref[...])
    z = _sigmoid(xs + xt)
    xmid = x + z * x2 + (1.0 - z) * yn
    xmid_ref[...] = xmid

    gid = gidr_ref[0]
    m = (lax.broadcasted_iota(jnp.int32, (G, B), 0) == gid).astype(jnp.float32)

    @pl.when(i == 0)
    def _():
        t1_ref[...] = jnp.zeros_like(t1_ref)
        t2_ref[...] = jnp.zeros_like(t2_ref)

    t1_ref[...] += jnp.dot(m, xmid, precision=_HIGH)
    t2_ref[...] += jnp.dot(m, xmid * xmid, precision=_HIGH)


def _fuse_call(x, gid_row3, gid_col3, y_flat, s1, s2, c3, g1, b1, a1,
               ws, bs, wt, bt, G, B, NB, D):
    const2 = pl.BlockSpec((G, D), lambda i: (0, 0))
    prm = pl.BlockSpec((1, D), lambda i: (0, 0))
    wspec = pl.BlockSpec((D, D), lambda i: (0, 0))
    return pl.pallas_call(
        functools.partial(_fuse_body, G, NB),
        grid=(NB,),
        in_specs=[
            pl.BlockSpec((B, D), lambda i: (i, 0)),
            pl.BlockSpec((1, 1, B), lambda i: (i, 0, 0)),
            pl.BlockSpec((1, B, 1), lambda i: (i, 0, 0)),
            const2, const2, const2, const2,
            prm, prm, prm, wspec, prm, wspec, prm,
        ],
        out_specs=[
            pl.BlockSpec((B, D), lambda i: (i, 0)),
            pl.BlockSpec((G, D), lambda i: (0, 0)),
            pl.BlockSpec((G, D), lambda i: (0, 0)),
        ],
        out_shape=[
            jax.ShapeDtypeStruct((NB * B, D), jnp.float32),
            jax.ShapeDtypeStruct((G, D), jnp.float32),
            jax.ShapeDtypeStruct((G, D), jnp.float32),
        ],
    )(x, gid_row3, gid_col3, y_flat, s1, s2, c3, g1, b1, a1, ws, bs, wt, bt)


def _norm2_body(G, xmid_ref, gidc_ref, t1_ref, t2_ref, c3_ref,
                g2_ref, b2_ref, a2_ref, x2b_ref):
    xm = xmid_ref[...]
    B, D = xm.shape
    a2 = a2_ref[...]
    mean, rstd = _norm_fields(t1_ref[...], t2_ref[...], c3_ref[...], a2)
    tbl = jnp.concatenate([mean, rstd], axis=1)
    gidc = gidc_ref[0]
    mt = (gidc == lax.broadcasted_iota(jnp.int32, (B, G), 1)).astype(jnp.float32)
    nv = jnp.dot(mt, tbl, precision=_HIGH)
    x2b_ref[...] = (g2_ref[...] * (xm - a2 * nv[:, :D]) * nv[:, D:]
                    + b2_ref[...])


def _norm2_call(xmid, gid_col3, t1, t2, c3, g2, b2, a2, G, B, NB, D):
    const2 = pl.BlockSpec((G, D), lambda i: (0, 0))
    prm = pl.BlockSpec((1, D), lambda i: (0, 0))
    return pl.pallas_call(
        functools.partial(_norm2_body, G),
        grid=(NB,),
        in_specs=[
            pl.BlockSpec((B, D), lambda i: (i, 0)),
            pl.BlockSpec((1, B, 1), lambda i: (i, 0, 0)),
            const2, const2, const2, prm, prm, prm,
        ],
        out_specs=pl.BlockSpec((B, D), lambda i: (i, 0)),
        out_shape=jax.ShapeDtypeStruct((NB * B, D), jnp.float32),
    )(xmid, gid_col3, t1, t2, c3, g2, b2, a2)


def _edge_agg(x2b, src3, dst3, zeros, N, D, NCH, K):
    ZR = (N // _NS) // 8 * 8
    TAIL = N - _NS * ZR

    mesh = plsc.VectorSubcoreMesh(core_axis_name="c", subcore_axis_name="s")

    @functools.partial(
        pl.kernel,
        out_type=jax.ShapeDtypeStruct((_NC, N, D), jnp.float32),
        mesh=mesh,
        scratch_types=[
            pltpu.VMEM((NCH, K), jnp.int32),
            pltpu.VMEM((NCH, K), jnp.int32),
            pltpu.VMEM((K, D), jnp.float32),
            pltpu.VMEM_SHARED((N, D), jnp.float32),
            pltpu.SemaphoreType.DMA,
        ],
    )
    def sc_kernel(x2_hbm, src_hbm, dst_hbm, zero_hbm, out_hbm,
                  src_v, dst_v, rows_v, acc, sem):
        c = lax.axis_index("c")
        s = lax.axis_index("s")
        wid = s * _NC + c

        pltpu.sync_copy(zero_hbm.at[pl.ds(s * ZR, ZR)], acc.at[pl.ds(s * ZR, ZR)])

        @pl.when(s == _NS - 1)
        def _():
            pltpu.sync_copy(zero_hbm.at[pl.ds(_NS * ZR, TAIL)],
                            acc.at[pl.ds(_NS * ZR, TAIL)])

        pltpu.sync_copy(src_hbm.at[wid], src_v)
        pltpu.sync_copy(dst_hbm.at[wid], dst_v)
        plsc.subcore_barrier()

        def body(j, carry):
            pltpu.async_copy(x2_hbm.at[src_v.at[j]], rows_v, sem).wait()
            pltpu.sync_copy(rows_v, acc.at[dst_v.at[j]], add=True)
            return carry

        lax.fori_loop(0, NCH, body, 0)
        plsc.subcore_barrier()

        pltpu.sync_copy(acc.at[pl.ds(s * ZR, ZR)],
                        out_hbm.at[c, pl.ds(s * ZR, ZR)])

        @pl.when(s == _NS - 1)
        def _():
            pltpu.sync_copy(acc.at[pl.ds(_NS * ZR, TAIL)],
                            out_hbm.at[c, pl.ds(_NS * ZR, TAIL)])

    return sc_kernel(x2b, src3, dst3, zeros)


def _gin_body(G, NB, x2b_ref, agg0_ref, agg1_ref, xmid_ref, gidr_ref,
              wg_ref, bg_ref, smid_ref, c3_ref, xout_ref, x3_ref):
    i = pl.program_id(0)
    t = x2b_ref[...] + agg0_ref[0] + agg1_ref[0]
    B = t.shape[0]
    h = jnp.maximum(jnp.dot(t, wg_ref[...], precision=_HIGH) + bg_ref[...], 0.0)
    xout_ref[...] = xmid_ref[...] + h

    gid = gidr_ref[0]
    m = (lax.broadcasted_iota(jnp.int32, (G, B), 0) == gid).astype(jnp.float32)

    @pl.when(i == 0)
    def _():
        x3_ref[...] = jnp.zeros_like(x3_ref)

    x3_ref[...] += jnp.dot(m, h, precision=_HIGH)

    @pl.when(i == NB - 1)
    def _():
        cnt = jnp.maximum(c3_ref[...], 1.0)
        x3_ref[...] = (x3_ref[...] + smid_ref[...]) / cnt


def _gin_call(x2b, parts, xmid, gid_row3, wg, bg, smid, c3, G, B, NB, D):
    const2 = pl.BlockSpec((G, D), lambda i: (0, 0))
    return pl.pallas_call(
        functools.partial(_gin_body, G, NB),
        grid=(NB,),
        in_specs=[
            pl.BlockSpec((B, D), lambda i: (i, 0)),
            pl.BlockSpec((1, B, D), lambda i: (0, i, 0)),
            pl.BlockSpec((1, B, D), lambda i: (1, i, 0)),
            pl.BlockSpec((B, D), lambda i: (i, 0)),
            pl.BlockSpec((1, 1, B), lambda i: (i, 0, 0)),
            pl.BlockSpec((D, D), lambda i: (0, 0)),
            pl.BlockSpec((1, D), lambda i: (0, 0)),
            const2, const2,
        ],
        out_specs=[
            pl.BlockSpec((B, D), lambda i: (i, 0)),
            pl.BlockSpec((G, D), lambda i: (0, 0)),
        ],
        out_shape=[
            jax.ShapeDtypeStruct((NB * B, D), jnp.float32),
            jax.ShapeDtypeStruct((G, D), jnp.float32),
        ],
    )(x2b, parts, parts, xmid, gid_row3, wg, bg, smid, c3)


def _pick_block(n, cap):
    best = 8
    for b in range(8, cap + 1, 8):
        if n % b == 0:
            best = b
    return best


def kernel(y, x, edge_index, graph_ids, gamma1, beta1, alpha1,
           gamma2, beta2, alpha2, WS, bS, WT, bT, W_gin, b_gin):
    N, D = x.shape
    BS, SRC, _ = y.shape
    G = BS * SRC
    E = edge_index.shape[1]

    B = _pick_block(N, 2048)
    NB = N // B

    EPW = E // _NW
    K = _pick_block(EPW, 128)
    NCH = EPW // K

    y_flat = y.reshape(G, D)
    gid_row3 = graph_ids.reshape(NB, 1, B)
    gid_col3 = graph_ids.reshape(NB, B, 1)
    p = lambda v: v.reshape(1, D)

    s1, s2, c3 = _stats_call(x, gid_row3, G, B, NB, D)
    xmid, t1, t2 = _fuse_call(x, gid_row3, gid_col3, y_flat, s1, s2, c3,
                              p(gamma1), p(beta1), p(alpha1),
                              WS, p(bS), WT, p(bT), G, B, NB, D)
    x2b = _norm2_call(xmid, gid_col3, t1, t2, c3,
                      p(gamma2), p(beta2), p(alpha2), G, B, NB, D)

    src3 = edge_index[0].reshape(_NW, NCH, K)
    dst3 = edge_index[1].reshape(_NW, NCH, K)
    zeros = jnp.zeros_like(x2b)
    parts = _edge_agg(x2b, src3, dst3, zeros, N, D, NCH, K)

    xout, x3 = _gin_call(x2b, parts, xmid, gid_row3, W_gin, p(b_gin),
                         t1, c3, G, B, NB, D)
    return x3.reshape(BS, SRC, D), xout


# speedup vs baseline: 8.5396x; 1.2832x over previous
import functools

import jax
import jax.numpy as jnp
from jax import lax
from jax.experimental import pallas as pl
from jax.experimental.pallas import tpu as pltpu
from jax.experimental.pallas import tpu_sc as plsc

_HIGH = lax.Precision.HIGHEST
_EPS = 1e-6

_NC = 2
_NS = 16
_NW = _NC * _NS


def _leaky(t):
    return jnp.where(t >= 0, t, 0.01 * t)


def _sigmoid(t):
    return 1.0 / (1.0 + jnp.exp(-t))


def _stats_body(G, NB, x_ref, gidr_ref, s1_ref, s2_ref, c3_ref):
    i = pl.program_id(0)
    x = x_ref[...]
    B = x.shape[0]
    gid = gidr_ref[0]
    m = (lax.broadcasted_iota(jnp.int32, (G, B), 0) == gid).astype(jnp.float32)

    @pl.when(i == 0)
    def _():
        s1_ref[...] = jnp.zeros_like(s1_ref)
        s2_ref[...] = jnp.zeros_like(s2_ref)
        c3_ref[...] = jnp.zeros_like(c3_ref)

    s1_ref[...] += jnp.dot(m, x, precision=_HIGH)
    s2_ref[...] += jnp.dot(m, x * x, precision=_HIGH)
    c3_ref[...] += jnp.dot(m, jnp.ones_like(x), precision=_HIGH)


def _stats_call(x, gid_row3, G, B, NB, D):
    return pl.pallas_call(
        functools.partial(_stats_body, G, NB),
        grid=(NB,),
        in_specs=[
            pl.BlockSpec((B, D), lambda i: (i, 0)),
            pl.BlockSpec((1, 1, B), lambda i: (i, 0, 0)),
        ],
        out_specs=[
            pl.BlockSpec((G, D), lambda i: (0, 0)),
            pl.BlockSpec((G, D), lambda i: (0, 0)),
            pl.BlockSpec((G, D), lambda i: (0, 0)),
        ],
        out_shape=[jax.ShapeDtypeStruct((G, D), jnp.float32)] * 3,
    )(x, gid_row3)


def _norm_fields(s1, s2, c3, alpha):
    cnt = jnp.maximum(c3, 1.0)
    mean = s1 / cnt
    meansq = s2 / cnt
    var = meansq - (2.0 * alpha - alpha * alpha) * mean * mean
    rstd = lax.rsqrt(var + _EPS)
    return mean, rstd


def _fuse_body(G, NB, x_ref, gidr_ref, gidc_ref, y_ref, s1_ref, s2_ref, c3_ref,
               g1_ref, b1_ref, a1_ref, ws_ref, bs_ref, wt_ref, bt_ref,
               xmid_ref, t1_ref, t2_ref):
    i = pl.program_id(0)
    x = x_ref[...]
    B, D = x.shape
    a1 = a1_ref[...]
    mean, rstd = _norm_fields(s1_ref[...], s2_ref[...], c3_ref[...], a1)
    tbl = jnp.concatenate([mean, rstd, y_ref[...]], axis=1)
    gidc = gidc_ref[0]
    mt = (gidc == lax.broadcasted_iota(jnp.int32, (B, G), 1)).astype(jnp.float32)
    nv = jnp.dot(mt, tbl, precision=_HIGH)
    mean_n = nv[:, :D]
    rstd_n = nv[:, D:2 * D]
    yn = nv[:, 2 * D:]
    x2 = g1_ref[...] * (x - a1 * mean_n) * rstd_n + b1_ref[...]
    xs = _leaky(jnp.dot(x2, ws_ref[...], precision=_HIGH) + bs_ref[...])
    xt = _leaky(jnp.dot(yn, wt_ref[...], precision=_HIGH) + bt_ref[...])
    z = _sigmoid(xs + xt)
    xmid = x + z * x2 + (1.0 - z) * yn
    xmid_ref[...] = xmid

    gid = gidr_ref[0]
    m = (lax.broadcasted_iota(jnp.int32, (G, B), 0) == gid).astype(jnp.float32)

    @pl.when(i == 0)
    def _():
        t1_ref[...] = jnp.zeros_like(t1_ref)
        t2_ref[...] = jnp.zeros_like(t2_ref)

    t1_ref[...] += jnp.dot(m, xmid, precision=_HIGH)
    t2_ref[...] += jnp.dot(m, xmid * xmid, precision=_HIGH)


def _fuse_call(x, gid_row3, gid_col3, y_flat, s1, s2, c3, g1, b1, a1,
               ws, bs, wt, bt, G, B, NB, D):
    const2 = pl.BlockSpec((G, D), lambda i: (0, 0))
    prm = pl.BlockSpec((1, D), lambda i: (0, 0))
    wspec = pl.BlockSpec((D, D), lambda i: (0, 0))
    return pl.pallas_call(
        functools.partial(_fuse_body, G, NB),
        grid=(NB,),
        in_specs=[
            pl.BlockSpec((B, D), lambda i: (i, 0)),
            pl.BlockSpec((1, 1, B), lambda i: (i, 0, 0)),
            pl.BlockSpec((1, B, 1), lambda i: (i, 0, 0)),
            const2, const2, const2, const2,
            prm, prm, prm, wspec, prm, wspec, prm,
        ],
        out_specs=[
            pl.BlockSpec((B, D), lambda i: (i, 0)),
            pl.BlockSpec((G, D), lambda i: (0, 0)),
            pl.BlockSpec((G, D), lambda i: (0, 0)),
        ],
        out_shape=[
            jax.ShapeDtypeStruct((NB * B, D), jnp.float32),
            jax.ShapeDtypeStruct((G, D), jnp.float32),
            jax.ShapeDtypeStruct((G, D), jnp.float32),
        ],
    )(x, gid_row3, gid_col3, y_flat, s1, s2, c3, g1, b1, a1, ws, bs, wt, bt)


def _norm2_body(G, xmid_ref, gidc_ref, t1_ref, t2_ref, c3_ref,
                g2_ref, b2_ref, a2_ref, x2b_ref):
    xm = xmid_ref[...]
    B, D = xm.shape
    a2 = a2_ref[...]
    mean, rstd = _norm_fields(t1_ref[...], t2_ref[...], c3_ref[...], a2)
    tbl = jnp.concatenate([mean, rstd], axis=1)
    gidc = gidc_ref[0]
    mt = (gidc == lax.broadcasted_iota(jnp.int32, (B, G), 1)).astype(jnp.float32)
    nv = jnp.dot(mt, tbl, precision=_HIGH)
    x2b_ref[...] = (g2_ref[...] * (xm - a2 * nv[:, :D]) * nv[:, D:]
                    + b2_ref[...])


def _norm2_call(xmid, gid_col3, t1, t2, c3, g2, b2, a2, G, B, NB, D):
    const2 = pl.BlockSpec((G, D), lambda i: (0, 0))
    prm = pl.BlockSpec((1, D), lambda i: (0, 0))
    return pl.pallas_call(
        functools.partial(_norm2_body, G),
        grid=(NB,),
        in_specs=[
            pl.BlockSpec((B, D), lambda i: (i, 0)),
            pl.BlockSpec((1, B, 1), lambda i: (i, 0, 0)),
            const2, const2, const2, prm, prm, prm,
        ],
        out_specs=pl.BlockSpec((B, D), lambda i: (i, 0)),
        out_shape=jax.ShapeDtypeStruct((NB * B, D), jnp.float32),
    )(xmid, gid_col3, t1, t2, c3, g2, b2, a2)


def _edge_agg(x2b, src4, dst4, zeros, N, D, SR, NSR, K):
    ZR = (N // _NS) // 8 * 8
    TAIL = N - _NS * ZR

    mesh = plsc.VectorSubcoreMesh(core_axis_name="c", subcore_axis_name="s")

    @functools.partial(
        pl.kernel,
        out_type=jax.ShapeDtypeStruct((_NC, N, D), jnp.float32),
        mesh=mesh,
        scratch_types=[
            pltpu.VMEM((SR, K), jnp.int32),
            pltpu.VMEM((SR, K), jnp.int32),
            pltpu.VMEM((K, D), jnp.float32),
            pltpu.VMEM((K, D), jnp.float32),
            pltpu.VMEM_SHARED((N, D), jnp.float32),
            pltpu.SemaphoreType.DMA,
            pltpu.SemaphoreType.DMA,
        ],
    )
    def sc_kernel(x2_hbm, src_hbm, dst_hbm, zero_hbm, out_hbm,
                  src_v, dst_v, rows0_v, rows1_v, acc, sem0, sem1):
        c = lax.axis_index("c")
        s = lax.axis_index("s")
        wid = s * _NC + c

        pltpu.sync_copy(zero_hbm.at[pl.ds(s * ZR, ZR)], acc.at[pl.ds(s * ZR, ZR)])

        @pl.when(s == _NS - 1)
        def _():
            pltpu.sync_copy(zero_hbm.at[pl.ds(_NS * ZR, TAIL)],
                            acc.at[pl.ds(_NS * ZR, TAIL)])

        plsc.subcore_barrier()

        def round_body(r, carry):
            pltpu.sync_copy(src_hbm.at[wid, r], src_v)
            pltpu.sync_copy(dst_hbm.at[wid, r], dst_v)
            pltpu.async_copy(x2_hbm.at[src_v.at[0]], rows0_v, sem0)

            def body(j, carry_in):
                b = lax.rem(j, 2)

                @pl.when(j + 1 < SR)
                def _():
                    @pl.when(b == 0)
                    def _():
                        pltpu.async_copy(x2_hbm.at[src_v.at[j + 1]],
                                         rows1_v, sem1)

                    @pl.when(b == 1)
                    def _():
                        pltpu.async_copy(x2_hbm.at[src_v.at[j + 1]],
                                         rows0_v, sem0)

                @pl.when(b == 0)
                def _():
                    pltpu.make_async_copy(x2_hbm.at[src_v.at[j]],
                                          rows0_v, sem0).wait()
                    pltpu.sync_copy(rows0_v, acc.at[dst_v.at[j]], add=True)

                @pl.when(b == 1)
                def _():
                    pltpu.make_async_copy(x2_hbm.at[src_v.at[j]],
                                          rows1_v, sem1).wait()
                    pltpu.sync_copy(rows1_v, acc.at[dst_v.at[j]], add=True)

                return carry_in

            return lax.fori_loop(0, SR, body, carry)

        lax.fori_loop(0, NSR, round_body, 0)
        plsc.subcore_barrier()

        pltpu.sync_copy(acc.at[pl.ds(s * ZR, ZR)],
                        out_hbm.at[c, pl.ds(s * ZR, ZR)])

        @pl.when(s == _NS - 1)
        def _():
            pltpu.sync_copy(acc.at[pl.ds(_NS * ZR, TAIL)],
                            out_hbm.at[c, pl.ds(_NS * ZR, TAIL)])

    return sc_kernel(x2b, src4, dst4, zeros)


def _gin_body(G, NB, x2b_ref, agg0_ref, agg1_ref, xmid_ref, gidr_ref,
              wg_ref, bg_ref, smid_ref, c3_ref, xout_ref, x3_ref):
    i = pl.program_id(0)
    t = x2b_ref[...] + agg0_ref[0] + agg1_ref[0]
    B = t.shape[0]
    h = jnp.maximum(jnp.dot(t, wg_ref[...], precision=_HIGH) + bg_ref[...], 0.0)
    xout_ref[...] = xmid_ref[...] + h

    gid = gidr_ref[0]
    m = (lax.broadcasted_iota(jnp.int32, (G, B), 0) == gid).astype(jnp.float32)

    @pl.when(i == 0)
    def _():
        x3_ref[...] = jnp.zeros_like(x3_ref)

    x3_ref[...] += jnp.dot(m, h, precision=_HIGH)

    @pl.when(i == NB - 1)
    def _():
        cnt = jnp.maximum(c3_ref[...], 1.0)
        x3_ref[...] = (x3_ref[...] + smid_ref[...]) / cnt


def _gin_call(x2b, parts, xmid, gid_row3, wg, bg, smid, c3, G, B, NB, D):
    const2 = pl.BlockSpec((G, D), lambda i: (0, 0))
    return pl.pallas_call(
        functools.partial(_gin_body, G, NB),
        grid=(NB,),
        in_specs=[
            pl.BlockSpec((B, D), lambda i: (i, 0)),
            pl.BlockSpec((1, B, D), lambda i: (0, i, 0)),
            pl.BlockSpec((1, B, D), lambda i: (1, i, 0)),
            pl.BlockSpec((B, D), lambda i: (i, 0)),
            pl.BlockSpec((1, 1, B), lambda i: (i, 0, 0)),
            pl.BlockSpec((D, D), lambda i: (0, 0)),
            pl.BlockSpec((1, D), lambda i: (0, 0)),
            const2, const2,
        ],
        out_specs=[
            pl.BlockSpec((B, D), lambda i: (i, 0)),
            pl.BlockSpec((G, D), lambda i: (0, 0)),
        ],
        out_shape=[
            jax.ShapeDtypeStruct((NB * B, D), jnp.float32),
            jax.ShapeDtypeStruct((G, D), jnp.float32),
        ],
    )(x2b, parts, parts, xmid, gid_row3, wg, bg, smid, c3)


def _pick_block(n, cap):
    best = 8
    for b in range(8, cap + 1, 8):
        if n % b == 0:
            best = b
    return best


def kernel(y, x, edge_index, graph_ids, gamma1, beta1, alpha1,
           gamma2, beta2, alpha2, WS, bS, WT, bT, W_gin, b_gin):
    N, D = x.shape
    BS, SRC, _ = y.shape
    G = BS * SRC
    E = edge_index.shape[1]

    B = _pick_block(N, 2048)
    NB = N // B

    EPW = E // _NW
    K = _pick_block(EPW, 128)
    NCH = EPW // K
    SR = 1
    for cand in range(1, NCH + 1):
        if NCH % cand == 0 and cand * K <= 2048:
            SR = cand
    NSR = NCH // SR

    y_flat = y.reshape(G, D)
    gid_row3 = graph_ids.reshape(NB, 1, B)
    gid_col3 = graph_ids.reshape(NB, B, 1)
    p = lambda v: v.reshape(1, D)

    s1, s2, c3 = _stats_call(x, gid_row3, G, B, NB, D)
    xmid, t1, t2 = _fuse_call(x, gid_row3, gid_col3, y_flat, s1, s2, c3,
                              p(gamma1), p(beta1), p(alpha1),
                              WS, p(bS), WT, p(bT), G, B, NB, D)
    x2b = _norm2_call(xmid, gid_col3, t1, t2, c3,
                      p(gamma2), p(beta2), p(alpha2), G, B, NB, D)

    src4 = edge_index[0].reshape(_NW, NSR, SR, K)
    dst4 = edge_index[1].reshape(_NW, NSR, SR, K)
    zeros = jnp.zeros_like(x2b)
    parts = _edge_agg(x2b, src4, dst4, zeros, N, D, SR, NSR, K)

    xout, x3 = _gin_call(x2b, parts, xmid, gid_row3, W_gin, p(b_gin),
                         t1, c3, G, B, NB, D)
    return x3.reshape(BS, SRC, D), xout


# speedup vs baseline: 8.6019x; 1.0073x over previous
import functools

import jax
import jax.numpy as jnp
from jax import lax
from jax.experimental import pallas as pl
from jax.experimental.pallas import tpu as pltpu
from jax.experimental.pallas import tpu_sc as plsc

_HIGH = lax.Precision.HIGHEST
_EPS = 1e-6

_NC = 2
_NS = 16
_NW = _NC * _NS


def _leaky(t):
    return jnp.where(t >= 0, t, 0.01 * t)


def _sigmoid(t):
    return 1.0 / (1.0 + jnp.exp(-t))


def _norm_fields(s1, s2, c3, alpha):
    cnt = jnp.maximum(c3, 1.0)
    mean = s1 / cnt
    meansq = s2 / cnt
    var = meansq - (2.0 * alpha - alpha * alpha) * mean * mean
    rstd = lax.rsqrt(var + _EPS)
    return mean, rstd


def _pre_body(G, NB, x_ref, gidr_ref, gidc_ref, y_ref,
              g1_ref, b1_ref, a1_ref, ws_ref, bs_ref, wt_ref, bt_ref,
              g2_ref, b2_ref, a2_ref,
              xmid_ref, x2b_ref, t1_ref, c3_ref,
              xmid_s, s1_s, s2_s, c3_s, t1_s, t2_s):
    i = pl.program_id(0)
    p = i // NB
    k = i % NB
    B = x_ref.shape[0]
    D = x_ref.shape[1]

    @pl.when(p == 0)
    def _():
        x = x_ref[...]
        gid = gidr_ref[0]
        m = (lax.broadcasted_iota(jnp.int32, (G, B), 0) == gid
             ).astype(jnp.float32)

        @pl.when(k == 0)
        def _():
            s1_s[...] = jnp.zeros_like(s1_s)
            s2_s[...] = jnp.zeros_like(s2_s)
            c3_s[...] = jnp.zeros_like(c3_s)

        s1_s[...] += jnp.dot(m, x, precision=_HIGH)
        s2_s[...] += jnp.dot(m, x * x, precision=_HIGH)
        c3_s[...] += jnp.dot(m, jnp.ones_like(x), precision=_HIGH)

    @pl.when(p == 1)
    def _():
        x = x_ref[...]
        a1 = a1_ref[...]
        mean, rstd = _norm_fields(s1_s[...], s2_s[...], c3_s[...], a1)
        tbl = jnp.concatenate([mean, rstd, y_ref[...]], axis=1)
        gidc = gidc_ref[0]
        mt = (gidc == lax.broadcasted_iota(jnp.int32, (B, G), 1)
              ).astype(jnp.float32)
        nv = jnp.dot(mt, tbl, precision=_HIGH)
        mean_n = nv[:, :D]
        rstd_n = nv[:, D:2 * D]
        yn = nv[:, 2 * D:]
        x2 = g1_ref[...] * (x - a1 * mean_n) * rstd_n + b1_ref[...]
        xs = _leaky(jnp.dot(x2, ws_ref[...], precision=_HIGH) + bs_ref[...])
        xt = _leaky(jnp.dot(yn, wt_ref[...], precision=_HIGH) + bt_ref[...])
        z = _sigmoid(xs + xt)
        xmid = x + z * x2 + (1.0 - z) * yn
        xmid_s[k] = xmid

        gid = gidr_ref[0]
        m = (lax.broadcasted_iota(jnp.int32, (G, B), 0) == gid
             ).astype(jnp.float32)

        @pl.when(k == 0)
        def _():
            t1_s[...] = jnp.zeros_like(t1_s)
            t2_s[...] = jnp.zeros_like(t2_s)

        t1_s[...] += jnp.dot(m, xmid, precision=_HIGH)
        t2_s[...] += jnp.dot(m, xmid * xmid, precision=_HIGH)

    @pl.when(p == 2)
    def _():
        xm = xmid_s[k]
        a2 = a2_ref[...]
        mean, rstd = _norm_fields(t1_s[...], t2_s[...], c3_s[...], a2)
        tbl = jnp.concatenate([mean, rstd], axis=1)
        gidc = gidc_ref[0]
        mt = (gidc == lax.broadcasted_iota(jnp.int32, (B, G), 1)
              ).astype(jnp.float32)
        nv = jnp.dot(mt, tbl, precision=_HIGH)
        xmid_ref[...] = xm
        x2b_ref[...] = (g2_ref[...] * (xm - a2 * nv[:, :D]) * nv[:, D:]
                        + b2_ref[...])

    @pl.when(i == 3 * NB - 1)
    def _():
        t1_ref[...] = t1_s[...]
        c3_ref[...] = c3_s[...]


def _pre_call(x, gid_row3, gid_col3, y_flat, g1, b1, a1, ws, bs, wt, bt,
              g2, b2, a2, G, B, NB, D):
    constg = pl.BlockSpec((G, D), lambda i: (0, 0))
    prm = pl.BlockSpec((1, D), lambda i: (0, 0))
    wspec = pl.BlockSpec((D, D), lambda i: (0, 0))
    return pl.pallas_call(
        functools.partial(_pre_body, G, NB),
        grid=(3 * NB,),
        in_specs=[
            pl.BlockSpec((B, D), lambda i: (i % NB, 0)),
            pl.BlockSpec((1, 1, B), lambda i: (i % NB, 0, 0)),
            pl.BlockSpec((1, B, 1), lambda i: (i % NB, 0, 0)),
            constg,
            prm, prm, prm, wspec, prm, wspec, prm, prm, prm, prm,
        ],
        out_specs=[
            pl.BlockSpec((B, D), lambda i: (jnp.maximum(i - 2 * NB, 0), 0)),
            pl.BlockSpec((B, D), lambda i: (jnp.maximum(i - 2 * NB, 0), 0)),
            constg, constg,
        ],
        out_shape=[
            jax.ShapeDtypeStruct((NB * B, D), jnp.float32),
            jax.ShapeDtypeStruct((NB * B, D), jnp.float32),
            jax.ShapeDtypeStruct((G, D), jnp.float32),
            jax.ShapeDtypeStruct((G, D), jnp.float32),
        ],
        scratch_shapes=[
            pltpu.VMEM((NB, B, D), jnp.float32),
            pltpu.VMEM((G, D), jnp.float32),
            pltpu.VMEM((G, D), jnp.float32),
            pltpu.VMEM((G, D), jnp.float32),
            pltpu.VMEM((G, D), jnp.float32),
            pltpu.VMEM((G, D), jnp.float32),
        ],
    )(x, gid_row3, gid_col3, y_flat, g1, b1, a1, ws, bs, wt, bt, g2, b2, a2)


def _edge_agg(x2b, src4, dst4, zeros, N, D, SR, NSR, K):
    ZR = (N // _NS) // 8 * 8
    TAIL = N - _NS * ZR

    mesh = plsc.VectorSubcoreMesh(core_axis_name="c", subcore_axis_name="s")

    @functools.partial(
        pl.kernel,
        out_type=jax.ShapeDtypeStruct((_NC, N, D), jnp.float32),
        mesh=mesh,
        scratch_types=[
            pltpu.VMEM((SR, K), jnp.int32),
            pltpu.VMEM((SR, K), jnp.int32),
            pltpu.VMEM((K, D), jnp.float32),
            pltpu.VMEM((K, D), jnp.float32),
            pltpu.VMEM_SHARED((N, D), jnp.float32),
            pltpu.SemaphoreType.DMA,
            pltpu.SemaphoreType.DMA,
        ],
    )
    def sc_kernel(x2_hbm, src_hbm, dst_hbm, zero_hbm, out_hbm,
                  src_v, dst_v, rows0_v, rows1_v, acc, sem0, sem1):
        c = lax.axis_index("c")
        s = lax.axis_index("s")
        wid = s * _NC + c

        pltpu.sync_copy(zero_hbm.at[pl.ds(s * ZR, ZR)], acc.at[pl.ds(s * ZR, ZR)])

        @pl.when(s == _NS - 1)
        def _():
            pltpu.sync_copy(zero_hbm.at[pl.ds(_NS * ZR, TAIL)],
                            acc.at[pl.ds(_NS * ZR, TAIL)])

        plsc.subcore_barrier()

        def round_body(r, carry):
            pltpu.sync_copy(src_hbm.at[wid, r], src_v)
            pltpu.sync_copy(dst_hbm.at[wid, r], dst_v)
            pltpu.async_copy(x2_hbm.at[src_v.at[0]], rows0_v, sem0)

            def body(j, carry_in):
                b = lax.rem(j, 2)

                @pl.when(j + 1 < SR)
                def _():
                    @pl.when(b == 0)
                    def _():
                        pltpu.async_copy(x2_hbm.at[src_v.at[j + 1]],
                                         rows1_v, sem1)

                    @pl.when(b == 1)
                    def _():
                        pltpu.async_copy(x2_hbm.at[src_v.at[j + 1]],
                                         rows0_v, sem0)

                @pl.when(b == 0)
                def _():
                    pltpu.make_async_copy(x2_hbm.at[src_v.at[j]],
                                          rows0_v, sem0).wait()
                    pltpu.sync_copy(rows0_v, acc.at[dst_v.at[j]], add=True)

                @pl.when(b == 1)
                def _():
                    pltpu.make_async_copy(x2_hbm.at[src_v.at[j]],
                                          rows1_v, sem1).wait()
                    pltpu.sync_copy(rows1_v, acc.at[dst_v.at[j]], add=True)

                return carry_in

            return lax.fori_loop(0, SR, body, carry)

        lax.fori_loop(0, NSR, round_body, 0)
        plsc.subcore_barrier()

        pltpu.sync_copy(acc.at[pl.ds(s * ZR, ZR)],
                        out_hbm.at[c, pl.ds(s * ZR, ZR)])

        @pl.when(s == _NS - 1)
        def _():
            pltpu.sync_copy(acc.at[pl.ds(_NS * ZR, TAIL)],
                            out_hbm.at[c, pl.ds(_NS * ZR, TAIL)])

    return sc_kernel(x2b, src4, dst4, zeros)


def _gin_body(G, NB, x2b_ref, agg0_ref, agg1_ref, xmid_ref, gidr_ref,
              wg_ref, bg_ref, smid_ref, c3_ref, xout_ref, x3_ref):
    i = pl.program_id(0)
    t = x2b_ref[...] + agg0_ref[0] + agg1_ref[0]
    B = t.shape[0]
    h = jnp.maximum(jnp.dot(t, wg_ref[...], precision=_HIGH) + bg_ref[...], 0.0)
    xout_ref[...] = xmid_ref[...] + h

    gid = gidr_ref[0]
    m = (lax.broadcasted_iota(jnp.int32, (G, B), 0) == gid).astype(jnp.float32)

    @pl.when(i == 0)
    def _():
        x3_ref[...] = jnp.zeros_like(x3_ref)

    x3_ref[...] += jnp.dot(m, h, precision=_HIGH)

    @pl.when(i == NB - 1)
    def _():
        cnt = jnp.maximum(c3_ref[...], 1.0)
        x3_ref[...] = (x3_ref[...] + smid_ref[...]) / cnt


def _gin_call(x2b, parts, xmid, gid_row3, wg, bg, smid, c3, G, B, NB, D):
    const2 = pl.BlockSpec((G, D), lambda i: (0, 0))
    return pl.pallas_call(
        functools.partial(_gin_body, G, NB),
        grid=(NB,),
        in_specs=[
            pl.BlockSpec((B, D), lambda i: (i, 0)),
            pl.BlockSpec((1, B, D), lambda i: (0, i, 0)),
            pl.BlockSpec((1, B, D), lambda i: (1, i, 0)),
            pl.BlockSpec((B, D), lambda i: (i, 0)),
            pl.BlockSpec((1, 1, B), lambda i: (i, 0, 0)),
            pl.BlockSpec((D, D), lambda i: (0, 0)),
            pl.BlockSpec((1, D), lambda i: (0, 0)),
            const2, const2,
        ],
        out_specs=[
            pl.BlockSpec((B, D), lambda i: (i, 0)),
            pl.BlockSpec((G, D), lambda i: (0, 0)),
        ],
        out_shape=[
            jax.ShapeDtypeStruct((NB * B, D), jnp.float32),
            jax.ShapeDtypeStruct((G, D), jnp.float32),
        ],
    )(x2b, parts, parts, xmid, gid_row3, wg, bg, smid, c3)


def _pick_block(n, cap):
    best = 8
    for b in range(8, cap + 1, 8):
        if n % b == 0:
            best = b
    return best


def kernel(y, x, edge_index, graph_ids, gamma1, beta1, alpha1,
           gamma2, beta2, alpha2, WS, bS, WT, bT, W_gin, b_gin):
    N, D = x.shape
    BS, SRC, _ = y.shape
    G = BS * SRC
    E = edge_index.shape[1]

    B = _pick_block(N, 2048)
    NB = N // B

    EPW = E // _NW
    K = _pick_block(EPW, 128)
    NCH = EPW // K
    SR = 1
    for cand in range(1, NCH + 1):
        if NCH % cand == 0 and cand * K <= 2048:
            SR = cand
    NSR = NCH // SR

    y_flat = y.reshape(G, D)
    gid_row3 = graph_ids.reshape(NB, 1, B)
    gid_col3 = graph_ids.reshape(NB, B, 1)
    p = lambda v: v.reshape(1, D)

    xmid, x2b, t1, c3 = _pre_call(x, gid_row3, gid_col3, y_flat,
                                  p(gamma1), p(beta1), p(alpha1),
                                  WS, p(bS), WT, p(bT),
                                  p(gamma2), p(beta2), p(alpha2),
                                  G, B, NB, D)

    src4 = edge_index[0].reshape(_NW, NSR, SR, K)
    dst4 = edge_index[1].reshape(_NW, NSR, SR, K)
    zeros = jnp.zeros_like(x2b)
    parts = _edge_agg(x2b, src4, dst4, zeros, N, D, SR, NSR, K)

    xout, x3 = _gin_call(x2b, parts, xmid, gid_row3, W_gin, p(b_gin),
                         t1, c3, G, B, NB, D)
    return x3.reshape(BS, SRC, D), xout


# speedup vs baseline: 11.8820x; 1.3813x over previous
import functools

import jax
import jax.numpy as jnp
from jax import lax
from jax.experimental import pallas as pl
from jax.experimental.pallas import tpu as pltpu
from jax.experimental.pallas import tpu_sc as plsc

_EPS = 1e-6

_NC = 2
_NS = 16
_NW = _NC * _NS


def _leaky(t):
    return jnp.where(t >= 0, t, 0.01 * t)


def _sigmoid(t):
    return 1.0 / (1.0 + jnp.exp(-t))


def _norm_fields(s1, s2, c3, alpha):
    cnt = jnp.maximum(c3, 1.0)
    mean = s1 / cnt
    meansq = s2 / cnt
    var = meansq - (2.0 * alpha - alpha * alpha) * mean * mean
    rstd = lax.rsqrt(var + _EPS)
    return mean, rstd


def _pre_body(G, NB, x_ref, gidr_ref, gidc_ref, y_ref,
              g1_ref, b1_ref, a1_ref, ws_ref, bs_ref, wt_ref, bt_ref,
              g2_ref, b2_ref, a2_ref,
              xmid_ref, x2b_ref, t1_ref, c3_ref,
              xmid_s, s1_s, s2_s, c3_s, t1_s, t2_s):
    i = pl.program_id(0)
    p = i // NB
    k = i % NB
    B = x_ref.shape[0]
    D = x_ref.shape[1]

    @pl.when(p == 0)
    def _():
        x = x_ref[...]
        gid = gidr_ref[0]
        m = (lax.broadcasted_iota(jnp.int32, (G, B), 0) == gid
             ).astype(jnp.float32)

        @pl.when(k == 0)
        def _():
            s1_s[...] = jnp.zeros_like(s1_s)
            s2_s[...] = jnp.zeros_like(s2_s)
            c3_s[...] = jnp.zeros_like(c3_s)

        s1_s[...] += jnp.dot(m, x)
        s2_s[...] += jnp.dot(m, x * x)
        c3_s[...] += jnp.dot(m, jnp.ones_like(x))

    @pl.when(p == 1)
    def _():
        x = x_ref[...]
        a1 = a1_ref[...]
        mean, rstd = _norm_fields(s1_s[...], s2_s[...], c3_s[...], a1)
        tbl = jnp.concatenate([mean, rstd, y_ref[...]], axis=1)
        gidc = gidc_ref[0]
        mt = (gidc == lax.broadcasted_iota(jnp.int32, (B, G), 1)
              ).astype(jnp.float32)
        nv = jnp.dot(mt, tbl)
        mean_n = nv[:, :D]
        rstd_n = nv[:, D:2 * D]
        yn = nv[:, 2 * D:]
        x2 = g1_ref[...] * (x - a1 * mean_n) * rstd_n + b1_ref[...]
        xs = _leaky(jnp.dot(x2, ws_ref[...]) + bs_ref[...])
        xt = _leaky(jnp.dot(yn, wt_ref[...]) + bt_ref[...])
        z = _sigmoid(xs + xt)
        xmid = x + z * x2 + (1.0 - z) * yn
        xmid_s[k] = xmid

        gid = gidr_ref[0]
        m = (lax.broadcasted_iota(jnp.int32, (G, B), 0) == gid
             ).astype(jnp.float32)

        @pl.when(k == 0)
        def _():
            t1_s[...] = jnp.zeros_like(t1_s)
            t2_s[...] = jnp.zeros_like(t2_s)

        t1_s[...] += jnp.dot(m, xmid)
        t2_s[...] += jnp.dot(m, xmid * xmid)

    @pl.when(p == 2)
    def _():
        xm = xmid_s[k]
        a2 = a2_ref[...]
        mean, rstd = _norm_fields(t1_s[...], t2_s[...], c3_s[...], a2)
        tbl = jnp.concatenate([mean, rstd], axis=1)
        gidc = gidc_ref[0]
        mt = (gidc == lax.broadcasted_iota(jnp.int32, (B, G), 1)
              ).astype(jnp.float32)
        nv = jnp.dot(mt, tbl)
        xmid_ref[...] = xm
        x2b_ref[...] = (g2_ref[...] * (xm - a2 * nv[:, :D]) * nv[:, D:]
                        + b2_ref[...])

    @pl.when(i == 3 * NB - 1)
    def _():
        t1_ref[...] = t1_s[...]
        c3_ref[...] = c3_s[...]


def _pre_call(x, gid_row3, gid_col3, y_flat, g1, b1, a1, ws, bs, wt, bt,
              g2, b2, a2, G, B, NB, D):
    constg = pl.BlockSpec((G, D), lambda i: (0, 0))
    prm = pl.BlockSpec((1, D), lambda i: (0, 0))
    wspec = pl.BlockSpec((D, D), lambda i: (0, 0))
    return pl.pallas_call(
        functools.partial(_pre_body, G, NB),
        grid=(3 * NB,),
        in_specs=[
            pl.BlockSpec((B, D), lambda i: (i % NB, 0)),
            pl.BlockSpec((1, 1, B), lambda i: (i % NB, 0, 0)),
            pl.BlockSpec((1, B, 1), lambda i: (i % NB, 0, 0)),
            constg,
            prm, prm, prm, wspec, prm, wspec, prm, prm, prm, prm,
        ],
        out_specs=[
            pl.BlockSpec((B, D), lambda i: (jnp.maximum(i - 2 * NB, 0), 0)),
            pl.BlockSpec((B, D), lambda i: (jnp.maximum(i - 2 * NB, 0), 0)),
            constg, constg,
        ],
        out_shape=[
            jax.ShapeDtypeStruct((NB * B, D), jnp.float32),
            jax.ShapeDtypeStruct((NB * B, D), jnp.float32),
            jax.ShapeDtypeStruct((G, D), jnp.float32),
            jax.ShapeDtypeStruct((G, D), jnp.float32),
        ],
        scratch_shapes=[
            pltpu.VMEM((NB, B, D), jnp.float32),
            pltpu.VMEM((G, D), jnp.float32),
            pltpu.VMEM((G, D), jnp.float32),
            pltpu.VMEM((G, D), jnp.float32),
            pltpu.VMEM((G, D), jnp.float32),
            pltpu.VMEM((G, D), jnp.float32),
        ],
    )(x, gid_row3, gid_col3, y_flat, g1, b1, a1, ws, bs, wt, bt, g2, b2, a2)


def _edge_agg(x2b, ei5, zeros, N, D, SR, NSR, K):
    ZR = (N // _NS) // 8 * 8
    TAIL = N - _NS * ZR

    mesh = plsc.VectorSubcoreMesh(core_axis_name="c", subcore_axis_name="s")

    @functools.partial(
        pl.kernel,
        out_type=jax.ShapeDtypeStruct((_NC, N, D), jnp.float32),
        mesh=mesh,
        scratch_types=[
            pltpu.VMEM((SR, K), jnp.int32),
            pltpu.VMEM((SR, K), jnp.int32),
            pltpu.VMEM((K, D), jnp.float32),
            pltpu.VMEM((K, D), jnp.float32),
            pltpu.VMEM_SHARED((N, D), jnp.float32),
            pltpu.SemaphoreType.DMA,
            pltpu.SemaphoreType.DMA,
        ],
    )
    def sc_kernel(x2_hbm, ei_hbm, zero_hbm, out_hbm,
                  src_v, dst_v, rows0_v, rows1_v, acc, sem0, sem1):
        c = lax.axis_index("c")
        s = lax.axis_index("s")
        wid = s * _NC + c

        pltpu.sync_copy(zero_hbm.at[pl.ds(s * ZR, ZR)], acc.at[pl.ds(s * ZR, ZR)])

        @pl.when(s == _NS - 1)
        def _():
            pltpu.sync_copy(zero_hbm.at[pl.ds(_NS * ZR, TAIL)],
                            acc.at[pl.ds(_NS * ZR, TAIL)])

        plsc.subcore_barrier()

        def round_body(r, carry):
            pltpu.sync_copy(ei_hbm.at[0, wid, r], src_v)
            pltpu.sync_copy(ei_hbm.at[1, wid, r], dst_v)
            pltpu.async_copy(x2_hbm.at[src_v.at[0]], rows0_v, sem0)

            def body(j, carry_in):
                b = lax.rem(j, 2)

                @pl.when(j + 1 < SR)
                def _():
                    @pl.when(b == 0)
                    def _():
                        pltpu.async_copy(x2_hbm.at[src_v.at[j + 1]],
                                         rows1_v, sem1)

                    @pl.when(b == 1)
                    def _():
                        pltpu.async_copy(x2_hbm.at[src_v.at[j + 1]],
                                         rows0_v, sem0)

                @pl.when(b == 0)
                def _():
                    pltpu.make_async_copy(x2_hbm.at[src_v.at[j]],
                                          rows0_v, sem0).wait()
                    pltpu.sync_copy(rows0_v, acc.at[dst_v.at[j]], add=True)

                @pl.when(b == 1)
                def _():
                    pltpu.make_async_copy(x2_hbm.at[src_v.at[j]],
                                          rows1_v, sem1).wait()
                    pltpu.sync_copy(rows1_v, acc.at[dst_v.at[j]], add=True)

                return carry_in

            return lax.fori_loop(0, SR, body, carry)

        lax.fori_loop(0, NSR, round_body, 0)
        plsc.subcore_barrier()

        pltpu.sync_copy(acc.at[pl.ds(s * ZR, ZR)],
                        out_hbm.at[c, pl.ds(s * ZR, ZR)])

        @pl.when(s == _NS - 1)
        def _():
            pltpu.sync_copy(acc.at[pl.ds(_NS * ZR, TAIL)],
                            out_hbm.at[c, pl.ds(_NS * ZR, TAIL)])

    return sc_kernel(x2b, ei5, zeros)


def _gin_body(G, NB, x2b_ref, agg0_ref, agg1_ref, xmid_ref, gidr_ref,
              wg_ref, bg_ref, smid_ref, c3_ref, xout_ref, x3_ref):
    i = pl.program_id(0)
    t = x2b_ref[...] + agg0_ref[0] + agg1_ref[0]
    B = t.shape[0]
    h = jnp.maximum(jnp.dot(t, wg_ref[...]) + bg_ref[...], 0.0)
    xout_ref[...] = xmid_ref[...] + h

    gid = gidr_ref[0]
    m = (lax.broadcasted_iota(jnp.int32, (G, B), 0) == gid).astype(jnp.float32)

    @pl.when(i == 0)
    def _():
        x3_ref[...] = jnp.zeros_like(x3_ref)

    x3_ref[...] += jnp.dot(m, h)

    @pl.when(i == NB - 1)
    def _():
        cnt = jnp.maximum(c3_ref[...], 1.0)
        x3_ref[...] = (x3_ref[...] + smid_ref[...]) / cnt


def _gin_call(x2b, parts, xmid, gid_row3, wg, bg, smid, c3, G, B, NB, D):
    const2 = pl.BlockSpec((G, D), lambda i: (0, 0))
    return pl.pallas_call(
        functools.partial(_gin_body, G, NB),
        grid=(NB,),
        in_specs=[
            pl.BlockSpec((B, D), lambda i: (i, 0)),
            pl.BlockSpec((1, B, D), lambda i: (0, i, 0)),
            pl.BlockSpec((1, B, D), lambda i: (1, i, 0)),
            pl.BlockSpec((B, D), lambda i: (i, 0)),
            pl.BlockSpec((1, 1, B), lambda i: (i, 0, 0)),
            pl.BlockSpec((D, D), lambda i: (0, 0)),
            pl.BlockSpec((1, D), lambda i: (0, 0)),
            const2, const2,
        ],
        out_specs=[
            pl.BlockSpec((B, D), lambda i: (i, 0)),
            pl.BlockSpec((G, D), lambda i: (0, 0)),
        ],
        out_shape=[
            jax.ShapeDtypeStruct((NB * B, D), jnp.float32),
            jax.ShapeDtypeStruct((G, D), jnp.float32),
        ],
    )(x2b, parts, parts, xmid, gid_row3, wg, bg, smid, c3)


def _pick_block(n, cap):
    best = 8
    for b in range(8, cap + 1, 8):
        if n % b == 0:
            best = b
    return best


def kernel(y, x, edge_index, graph_ids, gamma1, beta1, alpha1,
           gamma2, beta2, alpha2, WS, bS, WT, bT, W_gin, b_gin):
    N, D = x.shape
    BS, SRC, _ = y.shape
    G = BS * SRC
    E = edge_index.shape[1]

    B = _pick_block(N, 2048)
    NB = N // B

    EPW = E // _NW
    K = _pick_block(EPW, 128)
    NCH = EPW // K
    SR = 1
    for cand in range(1, NCH + 1):
        if NCH % cand == 0 and cand * K <= 2048:
            SR = cand
    NSR = NCH // SR

    y_flat = y.reshape(G, D)
    gid_row3 = graph_ids.reshape(NB, 1, B)
    gid_col3 = graph_ids.reshape(NB, B, 1)
    p = lambda v: v.reshape(1, D)

    xmid, x2b, t1, c3 = _pre_call(x, gid_row3, gid_col3, y_flat,
                                  p(gamma1), p(beta1), p(alpha1),
                                  WS, p(bS), WT, p(bT),
                                  p(gamma2), p(beta2), p(alpha2),
                                  G, B, NB, D)

    ei5 = edge_index.reshape(2, _NW, NSR, SR, K)
    zeros = jnp.zeros_like(x2b)
    parts = _edge_agg(x2b, ei5, zeros, N, D, SR, NSR, K)

    xout, x3 = _gin_call(x2b, parts, xmid, gid_row3, W_gin, p(b_gin),
                         t1, c3, G, B, NB, D)
    return x3.reshape(BS, SRC, D), xout
